# Initial kernel scaffold; baseline (speedup 1.0000x reference)
import jax
import jax.numpy as jnp
from jax.experimental import pallas as pl


def kernel(flux, time, band, band_table, flux_W, flux_b, W1, b1, W2, b2):
    raise NotImplementedError("write your pallas kernel here")


# speedup vs baseline: 3.2836x; 3.2836x over previous
import functools
import math

import jax
import jax.numpy as jnp
from jax.experimental import pallas as pl

_D = 32
_HALF = _D // 2
_PACK = 4
_LANES = _PACK * _D


def _fused_kernel(t_ref, f_ref, b_ref,
                  ang_w_ref, phase_ref, w1_ref, b1_ref, w2_ref,
                  fw_ref, bias_ref, e8_ref, kpat_ref, tmat_ref,
                  o_ref):
    f32 = jnp.float32
    t = t_ref[...]
    f = f_ref[...]
    bd = b_ref[...].astype(f32)

    ang = jnp.dot(t, ang_w_ref[...], preferred_element_type=f32) + phase_ref[...]
    se = jnp.sin(ang)

    h = jnp.dot(se, w1_ref[...], preferred_element_type=f32) + b1_ref[...]
    h = h * jax.nn.sigmoid(h)
    te = jnp.dot(h, w2_ref[...], preferred_element_type=f32)

    fe = jnp.dot(f, fw_ref[...], preferred_element_type=f32)

    bb = jnp.dot(bd, e8_ref[...], preferred_element_type=f32)
    oh = (bb == kpat_ref[...]).astype(f32)
    be = jnp.dot(oh, tmat_ref[...], preferred_element_type=f32)

    o_ref[...] = te + fe + be + bias_ref[...]


@functools.partial(jax.jit, static_argnames=())
def kernel(flux, time, band, band_table, flux_W, flux_b, W1, b1, W2, b2):
    B, L = flux.shape
    n = B * L
    n4 = n // _PACK
    f32 = jnp.float32
    eye4 = jnp.eye(_PACK, dtype=f32)

    freqs = jnp.exp(-math.log(10000.0) *
                    jnp.arange(_HALF, dtype=f32) / _HALF)
    freq32 = jnp.concatenate([freqs, freqs])
    ang_w = (eye4[:, :, None] * freq32[None, None, :]).reshape(_PACK, _LANES)
    phase = jnp.tile(
        jnp.concatenate([jnp.zeros((_HALF,), f32),
                         jnp.full((_HALF,), 0.5 * math.pi, f32)]),
        _PACK)[None, :]

    w1bd = (eye4[:, None, :, None] * W1[None, :, None, :]).reshape(_LANES, _LANES)
    w2bd = (eye4[:, None, :, None] * W2[None, :, None, :]).reshape(_LANES, _LANES)
    b1t = jnp.tile(b1, _PACK)[None, :]
    bias = jnp.tile(b2 + flux_b, _PACK)[None, :]

    fw = (eye4[:, :, None] * flux_W[:, 0][None, None, :]).reshape(_PACK, _LANES)

    e8 = (eye4[:, :, None] * jnp.ones((8,), f32)).reshape(_PACK, 32)
    kpat = jnp.tile(jnp.arange(8, dtype=f32), _PACK)[None, :]
    tpad = jnp.zeros((8, _D), f32).at[: band_table.shape[0]].set(band_table)
    tmat = (eye4[:, None, :, None] * tpad[None, :, None, :]).reshape(32, _LANES)

    t4 = time.reshape(n4, _PACK)
    f4 = flux.reshape(n4, _PACK)
    b4 = band.reshape(n4, _PACK)

    rows = 4096
    while n4 % rows:
        rows //= 2
    grid = (n4 // rows,)
    data_spec = pl.BlockSpec((rows, _PACK), lambda i: (i, 0))
    rep = lambda a: pl.BlockSpec(a.shape, lambda i: (0,) * a.ndim)

    out = pl.pallas_call(
        _fused_kernel,
        grid=grid,
        in_specs=[
            data_spec, data_spec, data_spec,
            rep(ang_w), rep(phase), rep(w1bd), rep(b1t), rep(w2bd),
            rep(fw), rep(bias), rep(e8), rep(kpat), rep(tmat),
        ],
        out_specs=pl.BlockSpec((rows, _LANES), lambda i: (i, 0)),
        out_shape=jax.ShapeDtypeStruct((n4, _LANES), f32),
    )(t4, f4, b4, ang_w, phase, w1bd, b1t, w2bd, fw, bias, e8, kpat, tmat)

    return out.reshape(B, L, _D)
